# baseline (speedup 1.0000x reference)
import functools

import jax
import jax.numpy as jnp
from jax import lax
from jax.experimental import pallas as pl
from jax.experimental.pallas import tpu as pltpu
from jax.experimental.pallas import tpu_sc as plsc

_B = 16384
_D = 20
_L = 16
_SC_B = 8192
_TC_BLK = 1024


def _sc_half(lengths, table):
    info = plsc.get_sparse_core_info()
    nw = info.num_cores * info.num_subcores
    b_per_w = _SC_B // nw
    mesh = plsc.VectorSubcoreMesh(core_axis_name="c", subcore_axis_name="s")

    @functools.partial(
        pl.kernel,
        mesh=mesh,
        out_type=jax.ShapeDtypeStruct((_SC_B, _D), jnp.float32),
        scratch_types=[
            pltpu.VMEM((b_per_w,), jnp.int32),
            pltpu.VMEM((10, _D), jnp.float32),
            pltpu.VMEM((b_per_w, _D), jnp.float32),
        ],
        compiler_params=pltpu.CompilerParams(needs_layout_passes=False),
    )
    def sc_kernel(lengths_hbm, table_hbm, out_hbm, len_v, table_v, out_v):
        wid = lax.axis_index("s") * info.num_cores + lax.axis_index("c")
        base = wid * b_per_w
        pltpu.sync_copy(lengths_hbm.at[pl.ds(base, b_per_w)], len_v)
        pltpu.sync_copy(table_hbm, table_v)

        lane = lax.iota(jnp.int32, _L)
        zero = lane * 0
        cols = [zero + c for c in range(_D)]

        @plsc.parallel_loop(0, b_per_w // _L, unroll=2)
        def body(c):
            lv = len_v[pl.ds(c * _L, _L)]
            bv = jnp.minimum(lv, 5)
            for t in (8, 16, 32, 64):
                bv = bv + jnp.where(lv >= t, 1, 0).astype(jnp.int32)
            rows16 = lane + c * _L
            for col in range(_D):
                vals = plsc.load_gather(table_v, [bv, cols[col]])
                plsc.store_scatter(out_v, [rows16, cols[col]], vals)

        pltpu.sync_copy(out_v, out_hbm.at[pl.ds(base, b_per_w)])

    return sc_kernel(lengths, table)


def _tc_half(lengths_col, table):
    n = lengths_col.shape[0]

    def tc_body(len_ref, tab_ref, out_ref):
        lv = len_ref[...]
        bv = jnp.minimum(lv, 5)
        for t in (8, 16, 32, 64):
            bv = bv + jnp.where(lv >= t, 1, 0).astype(jnp.int32)
        acc = jnp.zeros((_TC_BLK, _D), jnp.float32)
        for r in range(10):
            acc = acc + jnp.where(bv == r, 1.0, 0.0) * tab_ref[r : r + 1, :]
        out_ref[...] = acc

    return pl.pallas_call(
        tc_body,
        grid=(n // _TC_BLK,),
        in_specs=[
            pl.BlockSpec((_TC_BLK, 1), lambda g: (g, 0)),
            pl.BlockSpec((10, _D), lambda g: (0, 0)),
        ],
        out_specs=pl.BlockSpec((_TC_BLK, _D), lambda g: (g, 0)),
        out_shape=jax.ShapeDtypeStruct((n, _D), jnp.float32),
    )(lengths_col, table)


def kernel(lengths, table):
    lengths = lengths.astype(jnp.int32)
    sc_out = _sc_half(lengths[:_SC_B], table)
    tc_out = _tc_half(lengths[_SC_B:].reshape(_B - _SC_B, 1), table)
    return jnp.concatenate([sc_out, tc_out], axis=0)


# speedup vs baseline: 1.1068x; 1.1068x over previous
import functools

import jax
import jax.numpy as jnp
from jax import lax
from jax.experimental import pallas as pl
from jax.experimental.pallas import tpu as pltpu
from jax.experimental.pallas import tpu_sc as plsc

_B = 16384
_D = 20
_L = 16


def kernel(lengths, table):
    lengths = lengths.astype(jnp.int32)
    info = plsc.get_sparse_core_info()
    nw = info.num_cores * info.num_subcores
    b_per_w = _B // nw
    mesh = plsc.VectorSubcoreMesh(core_axis_name="c", subcore_axis_name="s")

    @functools.partial(
        pl.kernel,
        mesh=mesh,
        out_type=jax.ShapeDtypeStruct((_B, _D), jnp.float32),
        scratch_types=[
            pltpu.VMEM((b_per_w,), jnp.int32),
            pltpu.VMEM((10, _D), jnp.float32),
            pltpu.VMEM((b_per_w, _D), jnp.float32),
            pltpu.SemaphoreType.DMA,
        ],
        compiler_params=pltpu.CompilerParams(needs_layout_passes=False),
    )
    def sc_kernel(lengths_hbm, table_hbm, out_hbm, len_v, table_v, out_v, sem):
        wid = lax.axis_index("s") * info.num_cores + lax.axis_index("c")
        base = wid * b_per_w
        pltpu.sync_copy(lengths_hbm.at[pl.ds(base, b_per_w)], len_v)
        pltpu.sync_copy(table_hbm, table_v)

        lane = lax.iota(jnp.int32, _L)
        zero = lane * 0
        cols = [zero + c for c in range(_D)]

        half = b_per_w // 2

        def make_body(lo):
            @plsc.parallel_loop(lo, lo + half // _L, unroll=2)
            def body(c):
                lv = len_v[pl.ds(c * _L, _L)]
                bv = jnp.minimum(lv, 5)
                for t in (8, 16, 32, 64):
                    bv = bv + jnp.where(lv >= t, 1, 0).astype(jnp.int32)
                rows16 = lane + c * _L
                for col in range(_D):
                    vals = plsc.load_gather(table_v, [bv, cols[col]])
                    plsc.store_scatter(out_v, [rows16, cols[col]], vals)

        make_body(0)
        pltpu.async_copy(
            out_v.at[pl.ds(0, half)], out_hbm.at[pl.ds(base, half)], sem
        )
        make_body(half // _L)
        pltpu.make_async_copy(
            out_v.at[pl.ds(0, half)], out_hbm.at[pl.ds(base, half)], sem
        ).wait()
        pltpu.sync_copy(
            out_v.at[pl.ds(half, half)], out_hbm.at[pl.ds(base + half, half)]
        )

    return sc_kernel(lengths, table)
